# baseline (speedup 1.0000x reference)
import functools

import jax
import jax.numpy as jnp
from jax import lax
from jax.experimental import pallas as pl
from jax.experimental.pallas import tpu as pltpu
from jax.experimental.pallas import tpu_sc as plsc

_S = 4096
_D = 4096
_R = 2816
_SC_ROWS = _S - _R
_NC = 2
_NS = 16
_NW = _NC * _NS
_ROWS_PER_W = _SC_ROWS // _NW
_CH = 4
_NCHUNK = _ROWS_PER_W // _CH
_LANES = 16
_UNROLL = 8
_TC_BLOCK_ROWS = 256

_mesh = plsc.VectorSubcoreMesh(core_axis_name="c", subcore_axis_name="s")

_VBUF = pltpu.VMEM((_CH, _D), jnp.float32)


@functools.partial(
    pl.kernel,
    mesh=_mesh,
    out_type=jax.ShapeDtypeStruct((_SC_ROWS, _D), jnp.float32),
    scratch_types=[
        _VBUF, _VBUF, _VBUF,
        _VBUF, _VBUF, _VBUF,
        pltpu.SemaphoreType.DMA,
        pltpu.SemaphoreType.DMA,
        pltpu.SemaphoreType.DMA,
        pltpu.SemaphoreType.DMA,
    ],
)
def _sc_add(x_hbm, p_hbm, o_hbm, xv0, pv0, ov0, xv1, pv1, ov1,
            in0, in1, out0, out1):
    wid = lax.axis_index("s") * _NC + lax.axis_index("c")
    src_base = _R + wid * _ROWS_PER_W
    dst_base = wid * _ROWS_PER_W
    xv = (xv0, xv1)
    pv = (pv0, pv1)
    ov = (ov0, ov1)
    ins = (in0, in1)
    outs = (out0, out1)

    def start_in(chunk, b):
        rb = src_base + chunk * _CH
        pltpu.async_copy(x_hbm.at[pl.ds(rb, _CH)], xv[b], ins[b])
        pltpu.async_copy(p_hbm.at[pl.ds(rb, _CH)], pv[b], ins[b])

    def wait_in(b):
        pltpu.make_async_copy(
            x_hbm.at[pl.ds(src_base, _CH)], xv[b], ins[b]).wait()
        pltpu.make_async_copy(
            p_hbm.at[pl.ds(src_base, _CH)], pv[b], ins[b]).wait()

    def start_out(chunk, b):
        rb = dst_base + chunk * _CH
        pltpu.async_copy(ov[b], o_hbm.at[pl.ds(rb, _CH)], outs[b])

    def wait_out(b):
        pltpu.make_async_copy(
            ov[b], o_hbm.at[pl.ds(dst_base, _CH)], outs[b]).wait()

    start_in(0, 0)
    start_in(1, 1)

    def group_body(g, carry):
        for b in range(2):
            chunk = 2 * g + b
            wait_in(b)

            @pl.when(chunk >= 2)
            def _():
                wait_out(b)

            for r in range(_CH):
                def vec_body(j, carry2):
                    c = j * (_LANES * _UNROLL)
                    for u in range(_UNROLL):
                        s = pl.ds(c + u * _LANES, _LANES)
                        ov[b][r, s] = xv[b][r, s] + pv[b][r, s]
                    return carry2

                lax.fori_loop(0, _D // (_LANES * _UNROLL), vec_body, 0)

            start_out(chunk, b)

            @pl.when(chunk + 2 < _NCHUNK)
            def _():
                start_in(chunk + 2, b)
        return carry

    lax.fori_loop(0, _NCHUNK // 2, group_body, 0)
    wait_out(0)
    wait_out(1)


def _tc_add_body(x_ref, p_ref, o_ref):
    o_ref[...] = x_ref[...] + p_ref[...]


def _tc_add(x, pos_table):
    spec = pl.BlockSpec((_TC_BLOCK_ROWS, _D), lambda i: (i, 0))
    return pl.pallas_call(
        _tc_add_body,
        grid=(_R // _TC_BLOCK_ROWS,),
        in_specs=[spec, spec],
        out_specs=spec,
        out_shape=jax.ShapeDtypeStruct((_S, _D), jnp.float32),
    )(x, pos_table)


def _merge_body(band_ref, full_ref, o_ref):
    o_ref[...] = band_ref[...]


def _merge(sc_band, tc_full):
    return pl.pallas_call(
        _merge_body,
        grid=(_SC_ROWS // _TC_BLOCK_ROWS,),
        in_specs=[
            pl.BlockSpec((_TC_BLOCK_ROWS, _D), lambda i: (i, 0)),
            pl.BlockSpec(memory_space=pl.ANY),
        ],
        out_specs=pl.BlockSpec(
            (_TC_BLOCK_ROWS, _D),
            lambda i: (i + _R // _TC_BLOCK_ROWS, 0)),
        out_shape=jax.ShapeDtypeStruct((_S, _D), jnp.float32),
        input_output_aliases={1: 0},
    )(sc_band, tc_full)


def kernel(x, pos_table):
    sc_band = _sc_add(x, pos_table)
    tc_full = _tc_add(x, pos_table)
    return _merge(sc_band, tc_full)


# speedup vs baseline: 1.1243x; 1.1243x over previous
import functools

import jax
import jax.numpy as jnp
from jax import lax
from jax.experimental import pallas as pl
from jax.experimental.pallas import tpu as pltpu
from jax.experimental.pallas import tpu_sc as plsc

_S = 4096
_D = 4096
_R = 3584
_SC_ROWS = _S - _R
_NC = 2
_NS = 16
_NW = _NC * _NS
_ROWS_PER_W = _SC_ROWS // _NW
_CH = 4
_NCHUNK = _ROWS_PER_W // _CH
_LANES = 16
_UNROLL = 8
_TC_BLOCK_ROWS = 256

_mesh = plsc.VectorSubcoreMesh(core_axis_name="c", subcore_axis_name="s")

_VBUF = pltpu.VMEM((_CH, _D), jnp.float32)


@functools.partial(
    pl.kernel,
    mesh=_mesh,
    out_type=jax.ShapeDtypeStruct((_SC_ROWS, _D), jnp.float32),
    scratch_types=[
        _VBUF, _VBUF, _VBUF,
        _VBUF, _VBUF, _VBUF,
        pltpu.SemaphoreType.DMA,
        pltpu.SemaphoreType.DMA,
        pltpu.SemaphoreType.DMA,
        pltpu.SemaphoreType.DMA,
    ],
)
def _sc_add(x_hbm, p_hbm, o_hbm, xv0, pv0, ov0, xv1, pv1, ov1,
            in0, in1, out0, out1):
    wid = lax.axis_index("s") * _NC + lax.axis_index("c")
    src_base = _R + wid * _ROWS_PER_W
    dst_base = wid * _ROWS_PER_W
    xv = (xv0, xv1)
    pv = (pv0, pv1)
    ov = (ov0, ov1)
    ins = (in0, in1)
    outs = (out0, out1)

    def start_in(chunk, b):
        rb = src_base + chunk * _CH
        pltpu.async_copy(x_hbm.at[pl.ds(rb, _CH)], xv[b], ins[b])
        pltpu.async_copy(p_hbm.at[pl.ds(rb, _CH)], pv[b], ins[b])

    def wait_in(b):
        pltpu.make_async_copy(
            x_hbm.at[pl.ds(src_base, _CH)], xv[b], ins[b]).wait()
        pltpu.make_async_copy(
            p_hbm.at[pl.ds(src_base, _CH)], pv[b], ins[b]).wait()

    def start_out(chunk, b):
        rb = dst_base + chunk * _CH
        pltpu.async_copy(ov[b], o_hbm.at[pl.ds(rb, _CH)], outs[b])

    def wait_out(b):
        pltpu.make_async_copy(
            ov[b], o_hbm.at[pl.ds(dst_base, _CH)], outs[b]).wait()

    start_in(0, 0)
    start_in(1, 1)

    def group_body(g, carry):
        for b in range(2):
            chunk = 2 * g + b
            wait_in(b)

            @pl.when(chunk >= 2)
            def _():
                wait_out(b)

            for r in range(_CH):
                def vec_body(j, carry2):
                    c = j * (_LANES * _UNROLL)
                    for u in range(_UNROLL):
                        s = pl.ds(c + u * _LANES, _LANES)
                        ov[b][r, s] = xv[b][r, s] + pv[b][r, s]
                    return carry2

                lax.fori_loop(0, _D // (_LANES * _UNROLL), vec_body, 0)

            start_out(chunk, b)

            @pl.when(chunk + 2 < _NCHUNK)
            def _():
                start_in(chunk + 2, b)
        return carry

    lax.fori_loop(0, _NCHUNK // 2, group_body, 0)
    wait_out(0)
    wait_out(1)


def _tc_add_body(x_ref, p_ref, o_ref):
    o_ref[...] = x_ref[...] + p_ref[...]


def _tc_add(x, pos_table):
    spec = pl.BlockSpec((_TC_BLOCK_ROWS, _D), lambda i: (i, 0))
    return pl.pallas_call(
        _tc_add_body,
        grid=(_R // _TC_BLOCK_ROWS,),
        in_specs=[spec, spec],
        out_specs=spec,
        out_shape=jax.ShapeDtypeStruct((_S, _D), jnp.float32),
    )(x, pos_table)


def _merge_body(band_ref, full_ref, o_ref):
    o_ref[...] = band_ref[...]


def _merge(sc_band, tc_full):
    return pl.pallas_call(
        _merge_body,
        grid=(_SC_ROWS // _TC_BLOCK_ROWS,),
        in_specs=[
            pl.BlockSpec((_TC_BLOCK_ROWS, _D), lambda i: (i, 0)),
            pl.BlockSpec(memory_space=pl.ANY),
        ],
        out_specs=pl.BlockSpec(
            (_TC_BLOCK_ROWS, _D),
            lambda i: (i + _R // _TC_BLOCK_ROWS, 0)),
        out_shape=jax.ShapeDtypeStruct((_S, _D), jnp.float32),
        input_output_aliases={1: 0},
    )(sc_band, tc_full)


def kernel(x, pos_table):
    sc_band = _sc_add(x, pos_table)
    tc_full = _tc_add(x, pos_table)
    return _merge(sc_band, tc_full)


# speedup vs baseline: 1.5331x; 1.3637x over previous
import jax
import jax.numpy as jnp
from jax.experimental import pallas as pl

_BLOCK_ROWS = 256


def _add_kernel(x_ref, p_ref, o_ref):
    o_ref[...] = x_ref[...] + p_ref[...]


def kernel(x, pos_table):
    seq_len, d = x.shape
    grid = (seq_len // _BLOCK_ROWS,)
    spec = pl.BlockSpec((_BLOCK_ROWS, d), lambda i: (i, 0))
    return pl.pallas_call(
        _add_kernel,
        grid=grid,
        in_specs=[spec, spec],
        out_specs=spec,
        out_shape=jax.ShapeDtypeStruct((seq_len, d), x.dtype),
    )(x, pos_table)
